# Initial kernel scaffold; baseline (speedup 1.0000x reference)
import jax
import jax.numpy as jnp
from jax.experimental import pallas as pl


def kernel(x, codebooks):
    raise NotImplementedError("write your pallas kernel here")


# speedup vs baseline: 1.2987x; 1.2987x over previous
import functools

import jax
import jax.numpy as jnp
import numpy as np
from jax.experimental import pallas as pl

_B, _C2, _N = 16, 128, 1024
_C = _C2 // 2
_Q, _K, _D = 4, 1024, 64
_NCH = 512


def _rvq_kernel(x_ref, noise_ref, cb_ref, out_ref):
    mean = x_ref[0, :_C, :]
    scale = x_ref[0, _C:, :]
    stdev = jax.nn.softplus(scale) + 0.0001
    r = noise_ref[0] * stdev + mean

    iota_k = jax.lax.broadcasted_iota(jnp.int32, (_K, _NCH), 0)
    acc = jnp.zeros((_C, _NCH), dtype=jnp.float32)
    for q in range(_Q):
        cb = cb_ref[q]
        cnorm = jnp.sum(cb * cb, axis=1, keepdims=True)
        rnorm = jnp.sum(r * r, axis=0, keepdims=True)
        d = (rnorm - 2.0 * jax.lax.dot_general(
            cb, r, (((1,), (0,)), ((), ())),
            preferred_element_type=jnp.float32)) + cnorm
        minv = jnp.min(d, axis=0, keepdims=True)
        idx = jnp.min(jnp.where(d == minv, iota_k, _K), axis=0,
                      keepdims=True)
        onehot = (iota_k == idx).astype(jnp.float32)
        quant = jax.lax.dot_general(
            cb, onehot, (((0,), (0,)), ((), ())),
            precision=jax.lax.Precision.HIGHEST,
            preferred_element_type=jnp.float32)
        acc = acc + quant
        r = r - quant
    out_ref[0] = acc


@jax.jit
def kernel(x, codebooks):
    noise = jax.random.normal(jax.random.key(42), (_B, _C, _N),
                              dtype=jnp.float32)
    grid = (_B, _N // _NCH)
    return pl.pallas_call(
        _rvq_kernel,
        grid=grid,
        in_specs=[
            pl.BlockSpec((1, _C2, _NCH), lambda b, j: (b, 0, j)),
            pl.BlockSpec((1, _C, _NCH), lambda b, j: (b, 0, j)),
            pl.BlockSpec((_Q, _K, _D), lambda b, j: (0, 0, 0)),
        ],
        out_specs=pl.BlockSpec((1, _C, _NCH), lambda b, j: (b, 0, j)),
        out_shape=jax.ShapeDtypeStruct((_B, _C, _N), jnp.float32),
    )(x, noise, codebooks)


# speedup vs baseline: 1.8473x; 1.4224x over previous
import functools

import jax
import jax.numpy as jnp
import numpy as np
from jax.experimental import pallas as pl

_B, _C2, _N = 16, 128, 1024
_C = _C2 // 2
_Q, _K, _D = 4, 1024, 64
_NCH = 512


def _rvq_kernel(x_ref, noise_ref, cb_ref, cbh_ref, cbm_ref, cbl_ref,
                out_ref):
    mean = x_ref[0, :_C, :]
    scale = x_ref[0, _C:, :]
    stdev = jax.nn.softplus(scale) + 0.0001
    r = noise_ref[0] * stdev + mean

    iota_k = jax.lax.broadcasted_iota(jnp.int32, (_K, _NCH), 0)
    acc = jnp.zeros((_C, _NCH), dtype=jnp.float32)
    for q in range(_Q):
        cb = cb_ref[q]
        cnorm = jnp.sum(cb * cb, axis=1, keepdims=True)
        rnorm = jnp.sum(r * r, axis=0, keepdims=True)
        d = (rnorm - 2.0 * jax.lax.dot_general(
            cb, r, (((1,), (0,)), ((), ())),
            preferred_element_type=jnp.float32)) + cnorm
        minv = jnp.min(d, axis=0, keepdims=True)
        idx = jnp.min(jnp.where(d == minv, iota_k, _K), axis=0,
                      keepdims=True)
        onehot = (iota_k == idx).astype(jnp.bfloat16)
        dn = (((0,), (0,)), ((), ()))
        quant = (jax.lax.dot_general(
                     cbh_ref[q], onehot, dn,
                     preferred_element_type=jnp.float32)
                 + jax.lax.dot_general(
                     cbm_ref[q], onehot, dn,
                     preferred_element_type=jnp.float32)
                 + jax.lax.dot_general(
                     cbl_ref[q], onehot, dn,
                     preferred_element_type=jnp.float32))
        acc = acc + quant
        r = r - quant
    out_ref[0] = acc


@jax.jit
def kernel(x, codebooks):
    noise = jax.random.normal(jax.random.key(42), (_B, _C, _N),
                              dtype=jnp.float32)
    cb_hi = codebooks.astype(jnp.bfloat16)
    r1 = codebooks - cb_hi.astype(jnp.float32)
    cb_mid = r1.astype(jnp.bfloat16)
    cb_lo = (r1 - cb_mid.astype(jnp.float32)).astype(jnp.bfloat16)
    grid = (_B, _N // _NCH)
    cb_spec = pl.BlockSpec((_Q, _K, _D), lambda b, j: (0, 0, 0))
    return pl.pallas_call(
        _rvq_kernel,
        grid=grid,
        in_specs=[
            pl.BlockSpec((1, _C2, _NCH), lambda b, j: (b, 0, j)),
            pl.BlockSpec((1, _C, _NCH), lambda b, j: (b, 0, j)),
            cb_spec, cb_spec, cb_spec, cb_spec,
        ],
        out_specs=pl.BlockSpec((1, _C, _NCH), lambda b, j: (b, 0, j)),
        out_shape=jax.ShapeDtypeStruct((_B, _C, _N), jnp.float32),
    )(x, noise, codebooks, cb_hi, cb_mid, cb_lo)


# speedup vs baseline: 2.1710x; 1.1752x over previous
import jax
import jax.numpy as jnp
from jax.experimental import pallas as pl

_B, _C2, _N = 16, 128, 1024
_C = _C2 // 2
_Q, _K, _D = 4, 1024, 64
_NCH = 512
_SUB = 8


def _argmin_first(d):
    nch = d.shape[1]
    d3 = d.reshape(_K // _SUB, _SUB, nch)
    sub_iota = jax.lax.broadcasted_iota(jnp.int32, (_SUB, nch), 0)
    best = d3[0]
    bidx = sub_iota
    for i in range(1, _K // _SUB):
        v = d3[i]
        take = v < best
        best = jnp.where(take, v, best)
        bidx = jnp.where(take, sub_iota + (i * _SUB), bidx)
    n = _SUB
    while n > 1:
        h = n // 2
        av, bv = best[:h], best[h:n]
        ai, bi = bidx[:h], bidx[h:n]
        take = (bv < av) | ((bv == av) & (bi < ai))
        best = jnp.where(take, bv, av)
        bidx = jnp.where(take, bi, ai)
        n = h
    return bidx


def _rvq_kernel(x_ref, noise_ref, cbn_ref, cn_ref, cbh_ref, cbm_ref,
                cbl_ref, out_ref):
    mean = x_ref[0, :_C, :]
    scale = x_ref[0, _C:, :]
    stdev = jax.nn.softplus(scale) + 0.0001
    r = noise_ref[0] * stdev + mean

    iota_k = jax.lax.broadcasted_iota(jnp.int32, (_K, _NCH), 0)
    acc = jnp.zeros((_C, _NCH), dtype=jnp.float32)
    for q in range(_Q):
        rnorm = jnp.sum(r * r, axis=0, keepdims=True)
        d = (rnorm + jax.lax.dot_general(
            cbn_ref[q], r, (((1,), (0,)), ((), ())),
            preferred_element_type=jnp.float32)) + cn_ref[q]
        idx = _argmin_first(d)
        onehot = (iota_k == idx).astype(jnp.bfloat16)
        dn = (((0,), (0,)), ((), ()))
        quant = (jax.lax.dot_general(
                     cbh_ref[q], onehot, dn,
                     preferred_element_type=jnp.float32)
                 + jax.lax.dot_general(
                     cbm_ref[q], onehot, dn,
                     preferred_element_type=jnp.float32)
                 + jax.lax.dot_general(
                     cbl_ref[q], onehot, dn,
                     preferred_element_type=jnp.float32))
        acc = acc + quant
        r = r - quant
    out_ref[0] = acc


@jax.jit
def kernel(x, codebooks):
    noise = jax.random.normal(jax.random.key(42), (_B, _C, _N),
                              dtype=jnp.float32)
    cb_neg2 = -2.0 * codebooks
    cnorm = jnp.sum(codebooks * codebooks, axis=-1)[:, :, None]
    cb_hi = codebooks.astype(jnp.bfloat16)
    r1 = codebooks - cb_hi.astype(jnp.float32)
    cb_mid = r1.astype(jnp.bfloat16)
    cb_lo = (r1 - cb_mid.astype(jnp.float32)).astype(jnp.bfloat16)
    grid = (_B, _N // _NCH)
    cb_spec = pl.BlockSpec((_Q, _K, _D), lambda b, j: (0, 0, 0))
    return pl.pallas_call(
        _rvq_kernel,
        grid=grid,
        in_specs=[
            pl.BlockSpec((1, _C2, _NCH), lambda b, j: (b, 0, j)),
            pl.BlockSpec((1, _C, _NCH), lambda b, j: (b, 0, j)),
            cb_spec,
            pl.BlockSpec((_Q, _K, 1), lambda b, j: (0, 0, 0)),
            cb_spec, cb_spec, cb_spec,
        ],
        out_specs=pl.BlockSpec((1, _C, _NCH), lambda b, j: (b, 0, j)),
        out_shape=jax.ShapeDtypeStruct((_B, _C, _N), jnp.float32),
    )(x, noise, cb_neg2, cnorm, cb_hi, cb_mid, cb_lo)
